# Initial kernel scaffold; baseline (speedup 1.0000x reference)
import functools

import jax
import jax.numpy as jnp
from jax import lax
from jax.experimental import pallas as pl
from jax.experimental.pallas import tpu as pltpu
from jax.experimental.pallas import tpu_sc as plsc

ROW_BLOCK = 256


def _tc_body(nblocks, k, z_ref, cbt_ref, cw_ref, soft_ref, idx_ref, nw_ref,
             deadf_ref, hist_ref):
    i = pl.program_id(0)
    z = z_ref[...]
    cbt = cbt_ref[...]
    xc = lax.dot_general(z, cbt, (((1,), (0,)), ((), ())),
                         preferred_element_type=jnp.float32)
    x2 = jnp.sum(z * z, axis=1, keepdims=True)
    c2 = jnp.sum(cbt * cbt, axis=0, keepdims=True)
    dist = (x2 + c2) - 2.0 * xc

    t = -100.0 * dist
    m = jnp.max(t, axis=1, keepdims=True)
    e = jnp.exp(t - m)
    soft_ref[...] = e / jnp.sum(e, axis=1, keepdims=True)

    rowmin = jnp.min(dist, axis=1, keepdims=True)
    iota = lax.broadcasted_iota(jnp.int32, dist.shape, 1)
    idxv = jnp.min(jnp.where(dist == rowmin, iota, k), axis=1, keepdims=True)
    idx_ref[...] = idxv

    cnt = jnp.sum((idxv == iota).astype(jnp.int32), axis=0, keepdims=True)

    @pl.when(i == 0)
    def _():
        hist_ref[...] = cnt

    @pl.when(i > 0)
    def _():
        hist_ref[...] += cnt

    @pl.when(i == nblocks - 1)
    def _():
        nw = cw_ref[...] + hist_ref[...].astype(jnp.float32)
        nw_ref[...] = nw
        total = jnp.sum(nw)
        deadf_ref[...] = jnp.where(nw < total / (100.0 * k), 1.0, 0.0)


def _tc_call(z, cbt, cw, n, d, k):
    nblocks = n // ROW_BLOCK
    return pl.pallas_call(
        functools.partial(_tc_body, nblocks, k),
        grid=(nblocks,),
        in_specs=[
            pl.BlockSpec((ROW_BLOCK, d), lambda i: (i, 0)),
            pl.BlockSpec((d, k), lambda i: (0, 0)),
            pl.BlockSpec((1, k), lambda i: (0, 0)),
        ],
        out_specs=[
            pl.BlockSpec((ROW_BLOCK, k), lambda i: (i, 0)),
            pl.BlockSpec((ROW_BLOCK, 1), lambda i: (i, 0)),
            pl.BlockSpec((1, k), lambda i: (0, 0)),
            pl.BlockSpec((1, k), lambda i: (0, 0)),
        ],
        out_shape=[
            jax.ShapeDtypeStruct((n, k), jnp.float32),
            jax.ShapeDtypeStruct((n, 1), jnp.int32),
            jax.ShapeDtypeStruct((1, k), jnp.float32),
            jax.ShapeDtypeStruct((1, k), jnp.float32),
        ],
        scratch_shapes=[pltpu.VMEM((1, k), jnp.int32)],
    )(z, cbt, cw)


_NC, _NS, _L = 2, 16, 16
_NW = _NC * _NS
_CHUNK = 128


def _sc_body(n, k, d, idx_hbm, cb_hbm, z_hbm, ridx_hbm, deadf_hbm,
             quant_hbm, ul_hbm,
             idx_v, rows_a, rows_b, ridx_v, rl_v, cbl_v, deadw_v,
             sem_a, sem_b):
    rpw = n // _NW
    nch = rpw // _CHUNK
    kpw = k // _NW
    wid = lax.axis_index("s") * _NC + lax.axis_index("c")
    base = wid * rpw
    kbase = wid * kpw

    for c in range(nch):
        pltpu.sync_copy(idx_hbm.at[pl.ds(base + c * _CHUNK, _CHUNK)],
                        idx_v.at[c])

    pltpu.sync_copy(ridx_hbm.at[pl.ds(kbase, kpw)], ridx_v)
    pltpu.sync_copy(cb_hbm.at[pl.ds(kbase, kpw)], cbl_v)
    pltpu.sync_copy(deadf_hbm.at[pl.ds(kbase, kpw)], deadw_v)
    rl_cp = pltpu.async_copy(z_hbm.at[ridx_v], rl_v, sem_b)

    bufs = (rows_a, rows_b)
    copies = [None] * nch
    copies[0] = pltpu.async_copy(cb_hbm.at[idx_v.at[0]], bufs[0], sem_a)
    for c in range(nch):
        if c + 1 < nch:
            copies[c + 1] = pltpu.async_copy(
                cb_hbm.at[idx_v.at[c + 1]], bufs[(c + 1) % 2], sem_a)
        copies[c].wait()
        pltpu.sync_copy(bufs[c % 2],
                        quant_hbm.at[pl.ds(base + c * _CHUNK, _CHUNK)])

    rl_cp.wait()

    def row_body(r, carry):
        dvec = plsc.load_gather(deadw_v, [jnp.full((_L,), r, jnp.int32)])
        for cc in range(d // _L):
            sl = pl.ds(cc * _L, _L)
            rl_v[r, sl] = (rl_v[r, sl] - cbl_v[r, sl]) * dvec
        return carry

    lax.fori_loop(0, kpw, row_body, 0)
    pltpu.sync_copy(rl_v, ul_hbm.at[pl.ds(kbase, kpw)])


def _sc_call(idx, cb, z, ridx, deadf, n, d, k):
    rpw = n // _NW
    kpw = k // _NW
    mesh = plsc.VectorSubcoreMesh(core_axis_name="c", subcore_axis_name="s")
    return pl.kernel(
        functools.partial(_sc_body, n, k, d),
        out_type=(jax.ShapeDtypeStruct((n, d), jnp.float32),
                  jax.ShapeDtypeStruct((k, d), jnp.float32)),
        mesh=mesh,
        scratch_types=[
            pltpu.VMEM((rpw // _CHUNK, _CHUNK), jnp.int32),
            pltpu.VMEM((_CHUNK, d), jnp.float32),
            pltpu.VMEM((_CHUNK, d), jnp.float32),
            pltpu.VMEM((kpw,), jnp.int32),
            pltpu.VMEM((kpw, d), jnp.float32),
            pltpu.VMEM((kpw, d), jnp.float32),
            pltpu.VMEM((kpw,), jnp.float32),
            pltpu.SemaphoreType.DMA,
            pltpu.SemaphoreType.DMA,
        ],
    )(idx, cb, z, ridx, deadf)


def kernel(continuous_latent, codebook, codebook_weights, rand_idx):
    b, a, d = continuous_latent.shape
    k = codebook.shape[0]
    n = b * a
    z = continuous_latent.reshape(n, d)
    cbt = codebook.T
    cw = codebook_weights.reshape(1, k)

    soft, idx2, nw2, deadf2 = _tc_call(z, cbt, cw, n, d, k)
    quant, useless = _sc_call(idx2.reshape(n), codebook, z, rand_idx,
                              deadf2.reshape(k), n, d, k)
    return (quant.reshape(b, a, d), soft.reshape(b, a, k),
            nw2.reshape(k), useless)


# speedup vs baseline: 1.6736x; 1.6736x over previous
import functools

import jax
import jax.numpy as jnp
from jax import lax
from jax.experimental import pallas as pl
from jax.experimental.pallas import tpu as pltpu
from jax.experimental.pallas import tpu_sc as plsc

ROW_BLOCK = 256


def _tc_body(nblocks, k, z_ref, cbt_ref, cw_ref, soft_ref, idx_ref, nw_ref,
             deadm_ref, hist_ref):
    i = pl.program_id(0)
    z = z_ref[...]
    cbt = cbt_ref[...]
    xc = lax.dot_general(z, cbt, (((1,), (0,)), ((), ())),
                         preferred_element_type=jnp.float32)
    x2 = jnp.sum(z * z, axis=1, keepdims=True)
    c2 = jnp.sum(cbt * cbt, axis=0, keepdims=True)
    dist = (x2 + c2) - 2.0 * xc

    t = -100.0 * dist
    m = jnp.max(t, axis=1, keepdims=True)
    e = jnp.exp(t - m)
    soft_ref[...] = e / jnp.sum(e, axis=1, keepdims=True)

    rowmin = jnp.min(dist, axis=1, keepdims=True)
    iota = lax.broadcasted_iota(jnp.int32, dist.shape, 1)
    idxv = jnp.min(jnp.where(dist == rowmin, iota, k), axis=1, keepdims=True)
    idx_ref[...] = idxv

    cnt = jnp.sum((idxv == iota).astype(jnp.int32), axis=0, keepdims=True)

    @pl.when(i == 0)
    def _():
        hist_ref[...] = cnt

    @pl.when(i > 0)
    def _():
        hist_ref[...] += cnt

    @pl.when(i == nblocks - 1)
    def _():
        nw = cw_ref[...] + hist_ref[...].astype(jnp.float32)
        nw_ref[...] = nw
        total = jnp.sum(nw)
        deadf = jnp.where(nw < total / (100.0 * k), 1.0, 0.0)
        deadm_ref[...] = jnp.broadcast_to(deadf.reshape(k, 1),
                                          deadm_ref.shape)


def _tc_call(z, cbt, cw, n, d, k):
    nblocks = n // ROW_BLOCK
    return pl.pallas_call(
        functools.partial(_tc_body, nblocks, k),
        grid=(nblocks,),
        in_specs=[
            pl.BlockSpec((ROW_BLOCK, d), lambda i: (i, 0)),
            pl.BlockSpec((d, k), lambda i: (0, 0)),
            pl.BlockSpec((1, k), lambda i: (0, 0)),
        ],
        out_specs=[
            pl.BlockSpec((ROW_BLOCK, k), lambda i: (i, 0)),
            pl.BlockSpec((ROW_BLOCK, 1), lambda i: (i, 0)),
            pl.BlockSpec((1, k), lambda i: (0, 0)),
            pl.BlockSpec((k, d), lambda i: (0, 0)),
        ],
        out_shape=[
            jax.ShapeDtypeStruct((n, k), jnp.float32),
            jax.ShapeDtypeStruct((n, 1), jnp.int32),
            jax.ShapeDtypeStruct((1, k), jnp.float32),
            jax.ShapeDtypeStruct((k, d), jnp.float32),
        ],
        scratch_shapes=[pltpu.VMEM((1, k), jnp.int32)],
    )(z, cbt, cw)


_NC, _NS, _L = 2, 16, 16
_NW = _NC * _NS
_CHUNK = 128


def _sc_body(n, k, d, idx_hbm, cb_hbm, z_hbm, ridx_hbm, deadm_hbm,
             quant_hbm, ul_hbm,
             idx_v, rows_a, rows_b, ridx_v, rl_v, cbl_v, dm_v,
             sem_a, sem_b):
    rpw = n // _NW
    nch = rpw // _CHUNK
    kpw = k // _NW
    wid = lax.axis_index("s") * _NC + lax.axis_index("c")
    base = wid * rpw
    kbase = wid * kpw

    for c in range(nch):
        pltpu.sync_copy(idx_hbm.at[pl.ds(base + c * _CHUNK, _CHUNK)],
                        idx_v.at[c])

    pltpu.sync_copy(ridx_hbm.at[pl.ds(kbase, kpw)], ridx_v)
    pltpu.sync_copy(cb_hbm.at[pl.ds(kbase, kpw)], cbl_v)
    pltpu.sync_copy(deadm_hbm.at[pl.ds(kbase, kpw)], dm_v)
    rl_cp = pltpu.async_copy(z_hbm.at[ridx_v], rl_v, sem_b)

    bufs = (rows_a, rows_b)
    copies = [None] * nch
    copies[0] = pltpu.async_copy(cb_hbm.at[idx_v.at[0]], bufs[0], sem_a)
    for c in range(nch):
        if c + 1 < nch:
            copies[c + 1] = pltpu.async_copy(
                cb_hbm.at[idx_v.at[c + 1]], bufs[(c + 1) % 2], sem_a)
        copies[c].wait()
        pltpu.sync_copy(bufs[c % 2],
                        quant_hbm.at[pl.ds(base + c * _CHUNK, _CHUNK)])

    rl_cp.wait()

    def row_body(r, carry):
        for cc in range(d // _L):
            sl = pl.ds(cc * _L, _L)
            rl_v[r, sl] = (rl_v[r, sl] - cbl_v[r, sl]) * dm_v[r, sl]
        return carry

    lax.fori_loop(0, kpw, row_body, 0)
    pltpu.sync_copy(rl_v, ul_hbm.at[pl.ds(kbase, kpw)])


def _sc_call(idx, cb, z, ridx, deadm, n, d, k):
    rpw = n // _NW
    kpw = k // _NW
    mesh = plsc.VectorSubcoreMesh(core_axis_name="c", subcore_axis_name="s")
    return pl.kernel(
        functools.partial(_sc_body, n, k, d),
        out_type=(jax.ShapeDtypeStruct((n, d), jnp.float32),
                  jax.ShapeDtypeStruct((k, d), jnp.float32)),
        mesh=mesh,
        scratch_types=[
            pltpu.VMEM((rpw // _CHUNK, _CHUNK), jnp.int32),
            pltpu.VMEM((_CHUNK, d), jnp.float32),
            pltpu.VMEM((_CHUNK, d), jnp.float32),
            pltpu.VMEM((kpw,), jnp.int32),
            pltpu.VMEM((kpw, d), jnp.float32),
            pltpu.VMEM((kpw, d), jnp.float32),
            pltpu.VMEM((kpw, d), jnp.float32),
            pltpu.SemaphoreType.DMA,
            pltpu.SemaphoreType.DMA,
        ],
    )(idx, cb, z, ridx, deadm)


def kernel(continuous_latent, codebook, codebook_weights, rand_idx):
    b, a, d = continuous_latent.shape
    k = codebook.shape[0]
    n = b * a
    z = continuous_latent.reshape(n, d)
    cbt = codebook.T
    cw = codebook_weights.reshape(1, k)

    soft, idx2, nw2, deadm = _tc_call(z, cbt, cw, n, d, k)
    quant, useless = _sc_call(idx2.reshape(n), codebook, z, rand_idx,
                              deadm, n, d, k)
    return (quant.reshape(b, a, d), soft.reshape(b, a, k),
            nw2.reshape(k), useless)


# speedup vs baseline: 1.6844x; 1.0064x over previous
import functools

import jax
import jax.numpy as jnp
from jax import lax
from jax.experimental import pallas as pl
from jax.experimental.pallas import tpu as pltpu
from jax.experimental.pallas import tpu_sc as plsc

ROW_BLOCK = 256


def _tc_body(nblocks, k, z_ref, cbt_ref, cw_ref, soft_ref, idx_ref, nw_ref,
             deadm_ref, hist_ref, c2_ref):
    i = pl.program_id(0)
    z = z_ref[...]
    cbt = cbt_ref[...]

    @pl.when(i == 0)
    def _():
        c2_ref[...] = jnp.sum(cbt * cbt, axis=0, keepdims=True)

    xc = lax.dot_general(z, cbt, (((1,), (0,)), ((), ())),
                         preferred_element_type=jnp.float32)
    x2 = jnp.sum(z * z, axis=1, keepdims=True)
    dist = (x2 + c2_ref[...]) - 2.0 * xc

    t = -100.0 * dist
    m = jnp.max(t, axis=1, keepdims=True)
    e = jnp.exp(t - m)
    soft_ref[...] = e / jnp.sum(e, axis=1, keepdims=True)

    rowmin = jnp.min(dist, axis=1, keepdims=True)
    iota = lax.broadcasted_iota(jnp.int32, dist.shape, 1)
    idxv = jnp.min(jnp.where(dist == rowmin, iota, k), axis=1, keepdims=True)
    idx_ref[...] = idxv

    cnt = jnp.sum((idxv == iota).astype(jnp.int32), axis=0, keepdims=True)

    @pl.when(i == 0)
    def _():
        hist_ref[...] = cnt

    @pl.when(i > 0)
    def _():
        hist_ref[...] += cnt

    @pl.when(i == nblocks - 1)
    def _():
        nw = cw_ref[...] + hist_ref[...].astype(jnp.float32)
        nw_ref[...] = nw
        total = jnp.sum(nw)
        deadf = jnp.where(nw < total / (100.0 * k), 1.0, 0.0)
        deadm_ref[...] = jnp.broadcast_to(deadf.reshape(k, 1),
                                          deadm_ref.shape)


def _tc_call(z, cbt, cw, n, d, k):
    nblocks = n // ROW_BLOCK
    return pl.pallas_call(
        functools.partial(_tc_body, nblocks, k),
        grid=(nblocks,),
        in_specs=[
            pl.BlockSpec((ROW_BLOCK, d), lambda i: (i, 0)),
            pl.BlockSpec((d, k), lambda i: (0, 0)),
            pl.BlockSpec((1, k), lambda i: (0, 0)),
        ],
        out_specs=[
            pl.BlockSpec((ROW_BLOCK, k), lambda i: (i, 0)),
            pl.BlockSpec((ROW_BLOCK, 1), lambda i: (i, 0)),
            pl.BlockSpec((1, k), lambda i: (0, 0)),
            pl.BlockSpec((k, d), lambda i: (0, 0)),
        ],
        out_shape=[
            jax.ShapeDtypeStruct((n, k), jnp.float32),
            jax.ShapeDtypeStruct((n, 1), jnp.int32),
            jax.ShapeDtypeStruct((1, k), jnp.float32),
            jax.ShapeDtypeStruct((k, d), jnp.float32),
        ],
        scratch_shapes=[pltpu.VMEM((1, k), jnp.int32),
                        pltpu.VMEM((1, k), jnp.float32)],
    )(z, cbt, cw)


_NC, _NS, _L = 2, 16, 16
_NW = _NC * _NS
_CHUNK = 128


def _sc_body(n, k, d, idx_hbm, cb_hbm, z_hbm, ridx_hbm, deadm_hbm,
             quant_hbm, ul_hbm,
             idx_v, rows_a, rows_b, rows_c, ridx_v, rl_v, cbl_v, dm_v,
             sem_g, sem_o, sem_r):
    rpw = n // _NW
    nch = rpw // _CHUNK
    kpw = k // _NW
    wid = lax.axis_index("s") * _NC + lax.axis_index("c")
    base = wid * rpw
    kbase = wid * kpw

    pltpu.sync_copy(idx_hbm.at[pl.ds(wid * nch, nch)], idx_v)

    pltpu.sync_copy(ridx_hbm.at[pl.ds(kbase, kpw)], ridx_v)
    pltpu.sync_copy(cb_hbm.at[pl.ds(kbase, kpw)], cbl_v)
    pltpu.sync_copy(deadm_hbm.at[pl.ds(kbase, kpw)], dm_v)
    rl_cp = pltpu.async_copy(z_hbm.at[ridx_v], rl_v, sem_r)

    bufs = (rows_a, rows_b, rows_c)
    gcp = [None] * nch
    ocp = [None] * nch
    gcp[0] = pltpu.async_copy(cb_hbm.at[idx_v.at[0]], bufs[0], sem_g)
    if nch > 1:
        gcp[1] = pltpu.async_copy(cb_hbm.at[idx_v.at[1]], bufs[1], sem_g)

    rl_cp.wait()

    def row_body(r, carry):
        for cc in range(d // _L):
            sl = pl.ds(cc * _L, _L)
            rl_v[r, sl] = (rl_v[r, sl] - cbl_v[r, sl]) * dm_v[r, sl]
        return carry

    lax.fori_loop(0, kpw, row_body, 0)
    ul_cp = pltpu.async_copy(rl_v, ul_hbm.at[pl.ds(kbase, kpw)], sem_r)

    waited = [False] * nch
    for c in range(nch):
        gcp[c].wait()
        ocp[c] = pltpu.async_copy(
            bufs[c % 3], quant_hbm.at[pl.ds(base + c * _CHUNK, _CHUNK)],
            sem_o)
        nxt = c + 2
        if nxt < nch:
            if nxt >= 3:
                ocp[nxt - 3].wait()
                waited[nxt - 3] = True
            gcp[nxt] = pltpu.async_copy(
                cb_hbm.at[idx_v.at[nxt]], bufs[nxt % 3], sem_g)
    for c in range(nch):
        if not waited[c]:
            ocp[c].wait()
    ul_cp.wait()


def _sc_call(idx, cb, z, ridx, deadm, n, d, k):
    rpw = n // _NW
    kpw = k // _NW
    mesh = plsc.VectorSubcoreMesh(core_axis_name="c", subcore_axis_name="s")
    return pl.kernel(
        functools.partial(_sc_body, n, k, d),
        out_type=(jax.ShapeDtypeStruct((n, d), jnp.float32),
                  jax.ShapeDtypeStruct((k, d), jnp.float32)),
        mesh=mesh,
        scratch_types=[
            pltpu.VMEM((rpw // _CHUNK, _CHUNK), jnp.int32),
            pltpu.VMEM((_CHUNK, d), jnp.float32),
            pltpu.VMEM((_CHUNK, d), jnp.float32),
            pltpu.VMEM((_CHUNK, d), jnp.float32),
            pltpu.VMEM((kpw,), jnp.int32),
            pltpu.VMEM((kpw, d), jnp.float32),
            pltpu.VMEM((kpw, d), jnp.float32),
            pltpu.VMEM((kpw, d), jnp.float32),
            pltpu.SemaphoreType.DMA,
            pltpu.SemaphoreType.DMA,
            pltpu.SemaphoreType.DMA,
        ],
    )(idx.reshape(n // _CHUNK, _CHUNK), cb, z, ridx, deadm)


def kernel(continuous_latent, codebook, codebook_weights, rand_idx):
    b, a, d = continuous_latent.shape
    k = codebook.shape[0]
    n = b * a
    z = continuous_latent.reshape(n, d)
    cbt = codebook.T
    cw = codebook_weights.reshape(1, k)

    soft, idx2, nw2, deadm = _tc_call(z, cbt, cw, n, d, k)
    quant, useless = _sc_call(idx2.reshape(n), codebook, z, rand_idx,
                              deadm, n, d, k)
    return (quant.reshape(b, a, d), soft.reshape(b, a, k),
            nw2.reshape(k), useless)


# speedup vs baseline: 1.9113x; 1.1347x over previous
import functools

import jax
import jax.numpy as jnp
from jax import lax
from jax.experimental import pallas as pl
from jax.experimental.pallas import tpu as pltpu
from jax.experimental.pallas import tpu_sc as plsc

ROW_BLOCK = 512


def _tc_body(nblocks, k, z_ref, cbt_ref, cw_ref, soft_ref, idx_ref, nw_ref,
             deadm_ref, hist_ref, c2_ref):
    i = pl.program_id(0)
    z = z_ref[...]
    cbt = cbt_ref[...]

    @pl.when(i == 0)
    def _():
        c2_ref[...] = jnp.sum(cbt * cbt, axis=0, keepdims=True)

    xc = lax.dot_general(z, cbt, (((1,), (0,)), ((), ())),
                         preferred_element_type=jnp.float32)
    x2 = jnp.sum(z * z, axis=1, keepdims=True)
    dist = (x2 + c2_ref[...]) - 2.0 * xc

    rowmin = jnp.min(dist, axis=1, keepdims=True)
    t = -100.0 * dist
    m = -100.0 * rowmin
    e = jnp.exp(t - m)
    soft_ref[...] = e / jnp.sum(e, axis=1, keepdims=True)

    iota = lax.broadcasted_iota(jnp.int32, dist.shape, 1)
    idxv = jnp.min(jnp.where(dist == rowmin, iota, k), axis=1, keepdims=True)
    idx_ref[...] = idxv

    cnt = jnp.sum((idxv == iota).astype(jnp.int32), axis=0, keepdims=True)

    @pl.when(i == 0)
    def _():
        hist_ref[...] = cnt

    @pl.when(i > 0)
    def _():
        hist_ref[...] += cnt

    @pl.when(i == nblocks - 1)
    def _():
        nw = cw_ref[...] + hist_ref[...].astype(jnp.float32)
        nw_ref[...] = nw
        total = jnp.sum(nw)
        deadf = jnp.where(nw < total / (100.0 * k), 1.0, 0.0)
        deadm_ref[...] = jnp.broadcast_to(deadf.reshape(k, 1),
                                          deadm_ref.shape)


def _tc_call(z, cbt, cw, n, d, k):
    nblocks = n // ROW_BLOCK
    return pl.pallas_call(
        functools.partial(_tc_body, nblocks, k),
        grid=(nblocks,),
        in_specs=[
            pl.BlockSpec((ROW_BLOCK, d), lambda i: (i, 0)),
            pl.BlockSpec((d, k), lambda i: (0, 0)),
            pl.BlockSpec((1, k), lambda i: (0, 0)),
        ],
        out_specs=[
            pl.BlockSpec((ROW_BLOCK, k), lambda i: (i, 0)),
            pl.BlockSpec((ROW_BLOCK, 1), lambda i: (i, 0)),
            pl.BlockSpec((1, k), lambda i: (0, 0)),
            pl.BlockSpec((k, d), lambda i: (0, 0)),
        ],
        out_shape=[
            jax.ShapeDtypeStruct((n, k), jnp.float32),
            jax.ShapeDtypeStruct((n, 1), jnp.int32),
            jax.ShapeDtypeStruct((1, k), jnp.float32),
            jax.ShapeDtypeStruct((k, d), jnp.float32),
        ],
        scratch_shapes=[pltpu.VMEM((1, k), jnp.int32),
                        pltpu.VMEM((1, k), jnp.float32)],
    )(z, cbt, cw)


_NC, _NS, _L = 2, 16, 16
_NW = _NC * _NS
_CHUNK = 128


def _sc_body(n, k, d, idx_hbm, cb_hbm, z_hbm, ridx_hbm, deadm_hbm,
             quant_hbm, ul_hbm,
             idx_v, rows_a, rows_b, rows_c, ridx_v, rl_v, cbl_v, dm_v,
             sem_g, sem_o, sem_r):
    rpw = n // _NW
    nch = rpw // _CHUNK
    kpw = k // _NW
    sid = lax.axis_index("s")
    wid = sid * _NC + lax.axis_index("c")
    base = wid * rpw
    kbase = wid * kpw

    with jax.named_scope("sc_stage"):
        pltpu.sync_copy(idx_hbm.at[pl.ds(wid * nch, nch)], idx_v)

        pltpu.sync_copy(ridx_hbm.at[pl.ds(kbase, kpw)], ridx_v)
        pltpu.sync_copy(cb_hbm.at[pl.ds(kbase, kpw)], cbl_v)
        pltpu.sync_copy(deadm_hbm.at[pl.ds(kbase, kpw)], dm_v)
        rl_cp = pltpu.async_copy(z_hbm.at[ridx_v], rl_v, sem_r)

    bufs = (rows_a, rows_b, rows_c)
    gcp = [None] * nch
    ocp = [None] * nch
    gcp[0] = pltpu.async_copy(cb_hbm.at[idx_v.at[0]], bufs[0], sem_g)
    if nch > 1:
        gcp[1] = pltpu.async_copy(cb_hbm.at[idx_v.at[1]], bufs[1], sem_g)

    with jax.named_scope("sc_ul"):
        rl_cp.wait()

        def row_body(r, carry):
            for cc in range(d // _L):
                sl = pl.ds(cc * _L, _L)
                rl_v[r, sl] = (rl_v[r, sl] - cbl_v[r, sl]) * dm_v[r, sl]
            return carry

        lax.fori_loop(0, kpw, row_body, 0)
        ul_cp = pltpu.async_copy(rl_v, ul_hbm.at[pl.ds(kbase, kpw)], sem_r)

    with jax.named_scope("sc_qgather"):
        waited = [False] * nch
        for c in range(nch):
            gcp[c].wait()
            ocp[c] = pltpu.async_copy(
                bufs[c % 3], quant_hbm.at[pl.ds(base + c * _CHUNK, _CHUNK)],
                sem_o)
            nxt = c + 2
            if nxt < nch:
                if nxt >= 3:
                    ocp[nxt - 3].wait()
                    waited[nxt - 3] = True
                gcp[nxt] = pltpu.async_copy(
                    cb_hbm.at[idx_v.at[nxt]], bufs[nxt % 3], sem_g)
        for c in range(nch):
            if not waited[c]:
                ocp[c].wait()
        ul_cp.wait()


def _sc_call(idx, cb, z, ridx, deadm, n, d, k):
    rpw = n // _NW
    kpw = k // _NW
    mesh = plsc.VectorSubcoreMesh(core_axis_name="c", subcore_axis_name="s")
    return pl.kernel(
        functools.partial(_sc_body, n, k, d),
        out_type=(jax.ShapeDtypeStruct((n, d), jnp.float32),
                  jax.ShapeDtypeStruct((k, d), jnp.float32)),
        mesh=mesh,
        scratch_types=[
            pltpu.VMEM((rpw // _CHUNK, _CHUNK), jnp.int32),
            pltpu.VMEM((_CHUNK, d), jnp.float32),
            pltpu.VMEM((_CHUNK, d), jnp.float32),
            pltpu.VMEM((_CHUNK, d), jnp.float32),
            pltpu.VMEM((kpw,), jnp.int32),
            pltpu.VMEM((kpw, d), jnp.float32),
            pltpu.VMEM((kpw, d), jnp.float32),
            pltpu.VMEM((kpw, d), jnp.float32),
            pltpu.SemaphoreType.DMA,
            pltpu.SemaphoreType.DMA,
            pltpu.SemaphoreType.DMA,
        ],
    )(idx.reshape(n // _CHUNK, _CHUNK), cb, z, ridx, deadm)


def kernel(continuous_latent, codebook, codebook_weights, rand_idx):
    b, a, d = continuous_latent.shape
    k = codebook.shape[0]
    n = b * a
    z = continuous_latent.reshape(n, d)
    cbt = codebook.T
    cw = codebook_weights.reshape(1, k)

    soft, idx2, nw2, deadm = _tc_call(z, cbt, cw, n, d, k)
    quant, useless = _sc_call(idx2.reshape(n), codebook, z, rand_idx,
                              deadm, n, d, k)
    return (quant.reshape(b, a, d), soft.reshape(b, a, k),
            nw2.reshape(k), useless)
